# baseline (speedup 1.0000x reference)
import functools

import jax
import jax.numpy as jnp
from jax import lax
from jax.experimental import pallas as pl
from jax.experimental.pallas import tpu as pltpu
from jax.experimental.pallas import tpu_sc as plsc

DIM = 64
NW = 32
CHUNK = 128
NROW = 1000000
NFULL = NROW // CHUNK
TAIL = NROW - NFULL * CHUNK
JFULL = NFULL // NW
EXTRA = NFULL - JFULL * NW
BSZ, SEQ = 4096, 200
TBLK = SEQ // 8
DEPTH = 4


def _sc_reorder(tab_t):
    mesh = plsc.VectorSubcoreMesh(core_axis_name="c", subcore_axis_name="s")

    @functools.partial(
        pl.kernel,
        mesh=mesh,
        out_type=jax.ShapeDtypeStruct((NFULL * DIM, CHUNK), jnp.float32),
        scratch_types=[pltpu.SemaphoreType.DMA],
        compiler_params=pltpu.CompilerParams(use_tc_tiling_on_sc=True),
    )
    def k(tab_hbm, out_hbm, sem):
        wid = lax.axis_index("s") * 2 + lax.axis_index("c")

        def fire(k_):
            pltpu.async_copy(
                tab_hbm.at[:, pl.ds(k_ * CHUNK, CHUNK)],
                out_hbm.at[pl.ds(k_ * DIM, DIM)],
                sem,
            )

        def drain_one():
            pltpu.make_async_copy(
                tab_hbm.at[:, pl.ds(0, CHUNK)],
                out_hbm.at[pl.ds(0, DIM)],
                sem,
            ).wait()

        nblk = JFULL + 1
        for d in range(DEPTH):
            fire(d * NW + wid)

        def body(j, carry):
            drain_one()
            nk = (j + DEPTH) * NW + wid

            @pl.when(nk < NFULL)
            def _():
                fire(nk)

            return carry

        lax.fori_loop(0, JFULL - DEPTH, body, 0)
        for _ in range(DEPTH):
            drain_one()

        @pl.when(wid < EXTRA)
        def _():
            fire(JFULL * NW + wid)
            drain_one()

    return k(tab_t)


def _sc_linearize(staged, tail):
    mesh = plsc.VectorSubcoreMesh(core_axis_name="c", subcore_axis_name="s")

    @functools.partial(
        pl.kernel,
        mesh=mesh,
        out_type=jax.ShapeDtypeStruct((NROW, DIM), jnp.float32),
        scratch_types=[
            pltpu.VMEM((2, DIM, CHUNK), jnp.float32),
            pltpu.VMEM((2, CHUNK, DIM), jnp.float32),
            pltpu.VMEM((TAIL, DIM), jnp.float32),
            pltpu.SemaphoreType.DMA,
            pltpu.SemaphoreType.DMA,
            pltpu.SemaphoreType.DMA,
            pltpu.SemaphoreType.DMA,
        ],
        compiler_params=pltpu.CompilerParams(
            use_tc_tiling_on_sc=False, needs_layout_passes=False
        ),
    )
    def k(st_hbm, tail_hbm, out_hbm, src_v, dst_v, tail_v, r0, r1, w0, w1):
        wid = lax.axis_index("s") * 2 + lax.axis_index("c")
        rsems = (r0, r1)
        wsems = (w0, w1)
        rowq = [lax.iota(jnp.int32, 16) + 16 * q for q in range(4)]

        def fire(b, k_):
            pltpu.async_copy(
                st_hbm.at[pl.ds(k_ * DIM, DIM)], src_v.at[b], rsems[b]
            )

        def drain_in(b):
            pltpu.make_async_copy(
                st_hbm.at[pl.ds(0, DIM)], src_v.at[b], rsems[b]
            ).wait()

        def fire_out(b, k_):
            pltpu.async_copy(
                dst_v.at[b], out_hbm.at[pl.ds(k_ * CHUNK, CHUNK)], wsems[b]
            )

        def drain_out(b):
            pltpu.make_async_copy(
                dst_v.at[b], out_hbm.at[pl.ds(0, CHUNK)], wsems[b]
            ).wait()

        def transpose(b):
            def trow(rr, c):
                col = jnp.full((16,), rr, jnp.int32)
                for q in range(4):
                    v = plsc.load_gather(src_v.at[b], [rowq[q], col])
                    dst_v[b, rr, pl.ds(16 * q, 16)] = v
                return c

            lax.fori_loop(0, CHUNK, trow, 0)

        fire(0, wid)
        fire(1, NW + wid)

        def body(g, carry):
            for b in range(2):
                j = 2 * g + b
                k_ = j * NW + wid
                drain_in(b)

                @pl.when(g > 0)
                def _():
                    drain_out(b)

                transpose(b)
                fire_out(b, k_)
                nk = (j + 2) * NW + wid

                @pl.when(nk < NFULL)
                def _():
                    fire(b, nk)

            return carry

        lax.fori_loop(0, JFULL // 2, body, 0)
        drain_out(0)
        drain_out(1)

        @pl.when(wid < EXTRA)
        def _():
            drain_in(0)
            transpose(0)
            pltpu.sync_copy(
                dst_v.at[0],
                out_hbm.at[pl.ds((JFULL * NW + wid) * CHUNK, CHUNK)],
            )

        @pl.when(wid == NW - 1)
        def _():
            pltpu.sync_copy(tail_hbm, tail_v)
            pltpu.sync_copy(tail_v, out_hbm.at[pl.ds(NFULL * CHUNK, TAIL)])

    return k(staged, tail)


def _sc_gather(table, idx3):
    mesh = plsc.VectorSubcoreMesh(core_axis_name="c", subcore_axis_name="s")

    @functools.partial(
        pl.kernel,
        mesh=mesh,
        out_type=jax.ShapeDtypeStruct(
            (SEQ, DIM // 8, NW, 8, CHUNK), jnp.float32
        ),
        scratch_types=[
            pltpu.VMEM((TBLK, 8, CHUNK), jnp.int32),
            pltpu.VMEM((2, CHUNK, DIM), jnp.float32),
            pltpu.VMEM((2, DIM, CHUNK), jnp.float32),
            pltpu.SemaphoreType.DMA,
            pltpu.SemaphoreType.DMA,
            pltpu.SemaphoreType.DMA,
            pltpu.SemaphoreType.DMA,
        ],
        compiler_params=pltpu.CompilerParams(
            use_tc_tiling_on_sc=False, needs_layout_passes=False
        ),
    )
    def k(tab_hbm, idx_hbm, out_hbm, idx_v, rows_v, tr_v, g0, g1, w0, w1):
        wid = lax.axis_index("s") * 2 + lax.axis_index("c")
        gsems = (g0, g1)
        wsems = (w0, w1)
        rowq = [lax.iota(jnp.int32, 16) + 16 * q for q in range(8)]

        pltpu.sync_copy(idx_hbm.at[wid], idx_v)

        def fire_gather(b, t):
            pltpu.async_copy(
                tab_hbm.at[idx_v.at[t // 8, t % 8]], rows_v.at[b], gsems[b]
            )

        def drain_gather(b):
            pltpu.make_async_copy(
                tab_hbm.at[idx_v.at[0, 0]], rows_v.at[b], gsems[b]
            ).wait()

        def transpose(b):
            def trow(c, cc):
                col = jnp.full((16,), c, jnp.int32)
                for q in range(8):
                    v = plsc.load_gather(rows_v.at[b], [rowq[q], col])
                    tr_v[b, c, pl.ds(16 * q, 16)] = v
                return cc

            lax.fori_loop(0, DIM, trow, 0)

        def fire_writes(b, t):
            for cb in range(DIM // 8):
                pltpu.async_copy(
                    tr_v.at[b, pl.ds(8 * cb, 8)],
                    out_hbm.at[t, cb, wid],
                    wsems[b],
                )

        def drain_writes(b):
            for _ in range(DIM // 8):
                pltpu.make_async_copy(
                    tr_v.at[b, pl.ds(0, 8)], out_hbm.at[0, 0, 0], wsems[b]
                ).wait()

        fire_gather(0, 0)
        fire_gather(1, 1)

        def body(h, carry):
            for b in range(2):
                t = 2 * h + b
                drain_gather(b)

                @pl.when(h > 0)
                def _():
                    drain_writes(b)

                transpose(b)
                fire_writes(b, t)

                @pl.when(t + 2 < SEQ)
                def _():
                    fire_gather(b, t + 2)

            return carry

        lax.fori_loop(0, SEQ // 2, body, 0)
        drain_writes(0)
        drain_writes(1)

    return k(table, idx3)


def kernel(token_ids, weight):
    tail = weight[NFULL * CHUNK :, :]
    staged = _sc_reorder(weight.T)
    table = _sc_linearize(staged, tail)
    idx3 = (
        token_ids.T.reshape(TBLK, 8, NW, CHUNK)
        .transpose(2, 0, 1, 3)
        .astype(jnp.int32)
    )
    out5 = _sc_gather(table, idx3)
    return out5.transpose(2, 4, 0, 1, 3).reshape(BSZ, SEQ, DIM)


# speedup vs baseline: 10.1935x; 10.1935x over previous
import functools

import jax
import jax.numpy as jnp
from jax import lax
from jax.experimental import pallas as pl
from jax.experimental.pallas import tpu as pltpu
from jax.experimental.pallas import tpu_sc as plsc

PDIM = 128
NW = 32
CHUNK = 128
NSUB = 2
NBUF = 2

NROW = 1000000
NFULL = NROW // CHUNK
TAIL = NROW - NFULL * CHUNK
GROUPS = (NFULL // NW) // NBUF
EXTRA = NFULL - (GROUPS * NBUF) * NW


def _sc_gather(table, idx):
    nsuper = idx.shape[1] // NSUB
    half = nsuper // NBUF
    mesh = plsc.VectorSubcoreMesh(core_axis_name="c", subcore_axis_name="s")

    @functools.partial(
        pl.kernel,
        mesh=mesh,
        out_type=jax.ShapeDtypeStruct(
            (NW, nsuper, NSUB, CHUNK, PDIM), jnp.float32
        ),
        scratch_types=[
            pltpu.VMEM((nsuper * NSUB, CHUNK), jnp.int32),
            pltpu.VMEM((NBUF, NSUB, CHUNK, PDIM), jnp.float32),
            pltpu.SemaphoreType.DMA,
            pltpu.SemaphoreType.DMA,
            pltpu.SemaphoreType.DMA,
            pltpu.SemaphoreType.DMA,
        ],
        compiler_params=pltpu.CompilerParams(use_tc_tiling_on_sc=True),
    )
    def k(table_hbm, idx_hbm, out_hbm, idx_v, rows_v, g0, g1, w0, w1):
        wid = lax.axis_index("s") * 2 + lax.axis_index("c")
        pltpu.sync_copy(idx_hbm.at[wid], idx_v)
        gsems = (g0, g1)
        wsems = (w0, w1)

        def fire(buf, s):
            for u in range(NSUB):
                pltpu.async_copy(
                    table_hbm.at[idx_v.at[s * NSUB + u]],
                    rows_v.at[buf, u],
                    gsems[buf],
                )

        def drain_gathers(buf):
            for u in range(NSUB):
                pltpu.make_async_copy(
                    table_hbm.at[idx_v.at[u]], rows_v.at[buf, u], gsems[buf]
                ).wait()

        def start_write(buf, s):
            pltpu.async_copy(rows_v.at[buf], out_hbm.at[wid, s], wsems[buf])

        def drain_write(buf):
            pltpu.make_async_copy(
                rows_v.at[buf], out_hbm.at[wid, 0], wsems[buf]
            ).wait()

        fire(0, 0)
        fire(1, 1)

        def body(h, carry):
            drain_gathers(0)
            start_write(0, NBUF * h)
            drain_gathers(1)
            start_write(1, NBUF * h + 1)

            @pl.when(h < half - 1)
            def _():
                drain_write(0)
                fire(0, NBUF * h + 2)
                drain_write(1)
                fire(1, NBUF * h + 3)

            return carry

        lax.fori_loop(0, half, body, 0)
        drain_write(0)
        drain_write(1)

    return k(table, idx)


def kernel(token_ids, weight):
    bsz, seq = token_ids.shape
    dim = weight.shape[1]
    total = bsz * seq
    nchunk = total // (NW * CHUNK)
    table = jnp.pad(weight, ((0, 0), (0, PDIM - dim)))
    idx = token_ids.reshape(NW, nchunk, CHUNK).astype(jnp.int32)
    out = _sc_gather(table, idx)
    out = out.reshape(total, PDIM)[:, :dim]
    return out.reshape(bsz, seq, dim)
